# baseline (speedup 1.0000x reference)
import jax
import jax.numpy as jnp
from jax import lax
from jax.experimental import pallas as pl
from jax.experimental.pallas import tpu as pltpu
from jax.experimental.pallas import tpu_sc as plsc

NUM_FIELDS = 26
VOCAB = 100000
EMB = 50
BATCH = 16384
NUM_FEAT = 13
HIDDEN = 128
OUT = 2

NF_G = NUM_FIELDS // 2
V_G = NF_G * VOCAB

NC = 2
NS = 16
NW = NC * NS
GATHER_W = 128
B_PER_TILE = BATCH // NW
SUBBLOCKS = B_PER_TILE // GATHER_W
N_TBLK = BATCH // GATHER_W


def _gather_body(proj_hbm, idx_hbm, out_hbm, idx_v, rows_v, sem):
    wid = lax.axis_index("s") * NC + lax.axis_index("c")
    tblk = wid * SUBBLOCKS
    pltpu.sync_copy(idx_hbm.at[pl.ds(tblk, SUBBLOCKS)], idx_v)

    def sub(s, carry):
        pltpu.async_copy(proj_hbm.at[idx_v.at[s, 0]], rows_v, sem).wait()
        copies = [
            pltpu.async_copy(proj_hbm.at[idx_v.at[s, f]], rows_v, sem,
                             add=True)
            for f in range(1, NF_G)
        ]
        for c in copies:
            c.wait()
        pltpu.sync_copy(rows_v,
                        out_hbm.at[pl.ds((tblk + s) * GATHER_W, GATHER_W)])
        return carry

    lax.fori_loop(0, SUBBLOCKS, sub, 0)


def _sc_gather_add(proj, idx3):
    mesh = plsc.VectorSubcoreMesh(core_axis_name="c", subcore_axis_name="s")
    return pl.kernel(
        _gather_body,
        out_type=jax.ShapeDtypeStruct((BATCH, HIDDEN), jnp.float32),
        mesh=mesh,
        scratch_types=[
            pltpu.VMEM((SUBBLOCKS, NF_G, GATHER_W), jnp.int32),
            pltpu.VMEM((GATHER_W, HIDDEN), jnp.float32),
            pltpu.SemaphoreType.DMA,
        ],
        compiler_params=pltpu.CompilerParams(use_tc_tiling_on_sc=False),
    )(proj, idx3)


def _mlp_body(ha_ref, hb_ref, xnum_ref, w1n_ref, b1_ref, w2_ref, b2_ref,
              out_ref):
    h = (ha_ref[...] + hb_ref[...]
         + jnp.dot(xnum_ref[...], w1n_ref[...],
                   preferred_element_type=jnp.float32))
    h = jnp.maximum(h + b1_ref[...], 0.0)
    out_ref[...] = jnp.dot(h, w2_ref[...],
                           preferred_element_type=jnp.float32) + b2_ref[...]


def _tc_mlp(ha, hb, x_num, W1n, b1, W2, b2, block_b):
    grid = (BATCH // block_b,)
    return pl.pallas_call(
        _mlp_body,
        grid=grid,
        in_specs=[
            pl.BlockSpec((block_b, HIDDEN), lambda i: (i, 0)),
            pl.BlockSpec((block_b, HIDDEN), lambda i: (i, 0)),
            pl.BlockSpec((block_b, NUM_FEAT), lambda i: (i, 0)),
            pl.BlockSpec((NUM_FEAT, HIDDEN), lambda i: (0, 0)),
            pl.BlockSpec((1, HIDDEN), lambda i: (0, 0)),
            pl.BlockSpec((HIDDEN, OUT), lambda i: (0, 0)),
            pl.BlockSpec((1, OUT), lambda i: (0, 0)),
        ],
        out_specs=pl.BlockSpec((block_b, OUT), lambda i: (i, 0)),
        out_shape=jax.ShapeDtypeStruct((BATCH, OUT), jnp.float32),
    )(ha, hb, x_num, W1n, b1, W2, b2)


def _proj_group(t_T_g, W1_3_g):
    proj = jnp.einsum("fex,feh->fxh", t_T_g, W1_3_g,
                      preferred_element_type=jnp.float32)
    return proj.reshape(V_G, HIDDEN)


def _idx_group(x_cat_g):
    idx = (x_cat_g.astype(jnp.int32)
           + (jnp.arange(NF_G, dtype=jnp.int32) * VOCAB)[None, :])
    return jnp.transpose(idx.reshape(N_TBLK, GATHER_W, NF_G),
                         (0, 2, 1))


def kernel(x_cat, x_num, tables, W1, b1, W2, b2):
    t_T = jnp.transpose(tables, (0, 2, 1))
    W1_3 = W1[:NUM_FIELDS * EMB].reshape(NUM_FIELDS, EMB, HIDDEN)

    proj_a = _proj_group(t_T[:NF_G], W1_3[:NF_G])
    idx_a = _idx_group(x_cat[:, :NF_G])
    hsum_a = _sc_gather_add(proj_a, idx_a)

    proj_b = _proj_group(t_T[NF_G:], W1_3[NF_G:])
    idx_b = _idx_group(x_cat[:, NF_G:])
    hsum_b = _sc_gather_add(proj_b, idx_b)

    return _tc_mlp(hsum_a, hsum_b, x_num, W1[NUM_FIELDS * EMB:],
                   b1.reshape(1, HIDDEN), W2, b2.reshape(1, OUT),
                   block_b=2048)


# speedup vs baseline: 1.0104x; 1.0104x over previous
import jax
import jax.numpy as jnp
from jax import lax
from jax.experimental import pallas as pl
from jax.experimental.pallas import tpu as pltpu
from jax.experimental.pallas import tpu_sc as plsc

NUM_FIELDS = 26
VOCAB = 100000
EMB = 50
BATCH = 16384
NUM_FEAT = 13
HIDDEN = 128
OUT = 2

V_TOT = NUM_FIELDS * VOCAB

NC = 2
NS = 16
NW = NC * NS
GATHER_W = 128
B_PER_TILE = BATCH // NW
SUBBLOCKS = B_PER_TILE // GATHER_W
N_TBLK = BATCH // GATHER_W


def _gather_body(proj_hbm, idx_hbm, out_hbm, idx_v, rows_v, sem):
    wid = lax.axis_index("s") * NC + lax.axis_index("c")
    tblk = wid * SUBBLOCKS
    pltpu.sync_copy(idx_hbm.at[pl.ds(tblk, SUBBLOCKS)], idx_v)

    def sub(s, carry):
        pltpu.async_copy(proj_hbm.at[idx_v.at[s, 0]], rows_v, sem).wait()
        copies = [
            pltpu.async_copy(proj_hbm.at[idx_v.at[s, f]], rows_v, sem,
                             add=True)
            for f in range(1, NUM_FIELDS)
        ]
        for c in copies:
            c.wait()
        pltpu.sync_copy(rows_v,
                        out_hbm.at[pl.ds((tblk + s) * GATHER_W, GATHER_W)])
        return carry

    lax.fori_loop(0, SUBBLOCKS, sub, 0)


def _sc_gather_add(proj, idx3):
    mesh = plsc.VectorSubcoreMesh(core_axis_name="c", subcore_axis_name="s")
    return pl.kernel(
        _gather_body,
        out_type=jax.ShapeDtypeStruct((BATCH, HIDDEN), jnp.float32),
        mesh=mesh,
        scratch_types=[
            pltpu.VMEM((SUBBLOCKS, NUM_FIELDS, GATHER_W), jnp.int32),
            pltpu.VMEM((GATHER_W, HIDDEN), jnp.float32),
            pltpu.SemaphoreType.DMA,
        ],
        compiler_params=pltpu.CompilerParams(use_tc_tiling_on_sc=False),
    )(proj, idx3)


def _mlp_body(hsum_ref, xnum_ref, w1n_ref, b1_ref, w2_ref, b2_ref, out_ref):
    h = hsum_ref[...] + jnp.dot(xnum_ref[...], w1n_ref[...],
                                preferred_element_type=jnp.float32)
    h = jnp.maximum(h + b1_ref[...], 0.0)
    out_ref[...] = jnp.dot(h, w2_ref[...],
                           preferred_element_type=jnp.float32) + b2_ref[...]


def _tc_mlp(hsum, x_num, W1n, b1, W2, b2, block_b):
    grid = (BATCH // block_b,)
    return pl.pallas_call(
        _mlp_body,
        grid=grid,
        in_specs=[
            pl.BlockSpec((block_b, HIDDEN), lambda i: (i, 0)),
            pl.BlockSpec((block_b, NUM_FEAT), lambda i: (i, 0)),
            pl.BlockSpec((NUM_FEAT, HIDDEN), lambda i: (0, 0)),
            pl.BlockSpec((1, HIDDEN), lambda i: (0, 0)),
            pl.BlockSpec((HIDDEN, OUT), lambda i: (0, 0)),
            pl.BlockSpec((1, OUT), lambda i: (0, 0)),
        ],
        out_specs=pl.BlockSpec((block_b, OUT), lambda i: (i, 0)),
        out_shape=jax.ShapeDtypeStruct((BATCH, OUT), jnp.float32),
    )(hsum, x_num, W1n, b1, W2, b2)


def kernel(x_cat, x_num, tables, W1, b1, W2, b2):
    t_T = jnp.transpose(tables, (0, 2, 1))
    W1_3 = W1[:NUM_FIELDS * EMB].reshape(NUM_FIELDS, EMB, HIDDEN)
    proj = jnp.einsum("fex,feh->fxh", t_T, W1_3,
                      preferred_element_type=jnp.float32)
    proj = proj.reshape(V_TOT, HIDDEN)

    idx = (x_cat.astype(jnp.int32)
           + (jnp.arange(NUM_FIELDS, dtype=jnp.int32) * VOCAB)[None, :])
    idx3 = jnp.transpose(idx.reshape(N_TBLK, GATHER_W, NUM_FIELDS),
                         (0, 2, 1))

    hsum = _sc_gather_add(proj, idx3)
    return _tc_mlp(hsum, x_num, W1[NUM_FIELDS * EMB:],
                   b1.reshape(1, HIDDEN), W2, b2.reshape(1, OUT),
                   block_b=2048)
